# Initial kernel scaffold; baseline (speedup 1.0000x reference)
import jax
import jax.numpy as jnp
from jax.experimental import pallas as pl


def kernel(hidden_states, router_w, router_b, gate_w, up_w, down_w):
    raise NotImplementedError("write your pallas kernel here")


# speedup vs baseline: 1.7376x; 1.7376x over previous
import functools

import jax
import jax.numpy as jnp
from jax.experimental import pallas as pl
from jax.experimental.pallas import tpu as pltpu

S, H, I, E = 2048, 1024, 2816, 8
IT = 256
N_IT = I // IT


def _moe_kernel(x_ref, rw_w_ref, rb_ref, g_ref, u_ref, d_ref,
                out_ref, loss_ref, rws_ref):
    e = pl.program_id(0)
    it = pl.program_id(1)

    @pl.when(jnp.logical_and(e == 0, it == 0))
    def _router():
        x = x_ref[...]
        logits = jax.lax.dot_general(
            x, rw_w_ref[...], (((1,), (1,)), ((), ())),
            preferred_element_type=jnp.float32) + rb_ref[0, :]
        m = jnp.max(logits, axis=-1, keepdims=True)
        ex = jnp.exp(logits - m)
        rw = ex / jnp.sum(ex, axis=-1, keepdims=True)
        rws_ref[...] = rw
        diff = rw - (1.0 / E)
        loss_ref[...] = (jnp.mean(diff * diff) * 0.01).reshape(1, 1)
        out_ref[...] = jnp.zeros(out_ref.shape, out_ref.dtype)

    x = x_ref[...].astype(jnp.bfloat16)
    g = g_ref[0].astype(jnp.bfloat16)
    u = u_ref[0].astype(jnp.bfloat16)
    dwn = d_ref[0].astype(jnp.bfloat16)
    gate = jax.lax.dot_general(x, g, (((1,), (1,)), ((), ())),
                               preferred_element_type=jnp.float32)
    up = jax.lax.dot_general(x, u, (((1,), (1,)), ((), ())),
                             preferred_element_type=jnp.float32)
    t = gate * jax.nn.sigmoid(gate) * up
    lane = jax.lax.broadcasted_iota(jnp.int32, (S, E), 1)
    w_e = jnp.sum(jnp.where(lane == e, rws_ref[...], 0.0), axis=1,
                  keepdims=True)
    t = (t * w_e).astype(jnp.bfloat16)
    out_ref[...] += jax.lax.dot_general(t, dwn, (((1,), (1,)), ((), ())),
                                        preferred_element_type=jnp.float32)


@functools.partial(jax.jit, static_argnames=())
def kernel(hidden_states, router_w, router_b, gate_w, up_w, down_w):
    x = hidden_states.reshape(S, H)
    rb = router_b.reshape(1, E)
    out, loss = pl.pallas_call(
        _moe_kernel,
        grid=(E, N_IT),
        in_specs=[
            pl.BlockSpec((S, H), lambda e, i: (0, 0)),
            pl.BlockSpec((E, H), lambda e, i: (0, 0)),
            pl.BlockSpec((1, E), lambda e, i: (0, 0)),
            pl.BlockSpec((1, IT, H), lambda e, i: (e, i, 0)),
            pl.BlockSpec((1, IT, H), lambda e, i: (e, i, 0)),
            pl.BlockSpec((1, H, IT), lambda e, i: (e, 0, i)),
        ],
        out_specs=[
            pl.BlockSpec((S, H), lambda e, i: (0, 0)),
            pl.BlockSpec((1, 1), lambda e, i: (0, 0)),
        ],
        out_shape=[
            jax.ShapeDtypeStruct((S, H), jnp.float32),
            jax.ShapeDtypeStruct((1, 1), jnp.float32),
        ],
        scratch_shapes=[pltpu.VMEM((S, E), jnp.float32)],
    )(x, router_w, rb, gate_w, up_w, down_w)
    return out.reshape(hidden_states.shape), loss[0, 0]
